# baseline (speedup 1.0000x reference)
import functools
import math

import jax
import jax.numpy as jnp
import numpy as np
from jax.experimental import pallas as pl
from jax.experimental.pallas import tpu as pltpu

B, S, D = 1, 2048, 1024
H = 16
HD = D // H
E = 8
DFF = 4 * D
CAP = math.ceil(1.25 * S / E)
MEM_LEN = 256
CMEM_LEN = 128
KF = 4
DFB = DFF // KF


def _mm(a, w, bias=None, *, ln=None, act=None, res=None, bm=256, bn=512):
    M, K = a.shape
    N = w.shape[1]
    bm = min(bm, M)
    bn = min(bn, N)
    operands = [a, w]
    specs = [
        pl.BlockSpec((bm, K), lambda i, j: (i, 0)),
        pl.BlockSpec((K, bn), lambda i, j: (0, j)),
    ]
    has_bias = bias is not None
    has_ln = ln is not None
    has_res = res is not None
    if has_bias:
        operands.append(bias.reshape(1, N))
        specs.append(pl.BlockSpec((1, bn), lambda i, j: (0, j)))
    if has_ln:
        g, be = ln
        operands += [g.reshape(1, K), be.reshape(1, K)]
        specs += [pl.BlockSpec((1, K), lambda i, j: (0, 0))] * 2
    if has_res:
        operands.append(res)
        specs.append(pl.BlockSpec((bm, bn), lambda i, j: (i, j)))

    def kfn(*refs):
        it = iter(refs)
        a_ref = next(it)
        w_ref = next(it)
        b_ref = next(it) if has_bias else None
        g_ref = next(it) if has_ln else None
        be_ref = next(it) if has_ln else None
        r_ref = next(it) if has_res else None
        o_ref = next(it)
        av = a_ref[...]
        if has_ln:
            mu = jnp.mean(av, axis=1, keepdims=True)
            var = jnp.mean((av - mu) ** 2, axis=1, keepdims=True)
            av = (av - mu) / jnp.sqrt(var + 1e-5) * g_ref[...] + be_ref[...]
        o = jnp.dot(av, w_ref[...], preferred_element_type=jnp.float32)
        if has_bias:
            o = o + b_ref[...]
        if act == "relu":
            o = jnp.maximum(o, 0.0)
        if has_res:
            o = o + r_ref[...]
        o_ref[...] = o

    return pl.pallas_call(
        kfn,
        grid=(M // bm, N // bn),
        in_specs=specs,
        out_specs=pl.BlockSpec((bm, bn), lambda i, j: (i, j)),
        out_shape=jax.ShapeDtypeStruct((M, N), jnp.float32),
    )(*operands)


def _attn_kernel(q_ref, k_ref, v_ref, o_ref, *, bq):
    i = pl.program_id(1)
    q = q_ref[0]
    s = jax.lax.dot_general(
        q, k_ref[0], (((1,), (1,)), ((), ())),
        preferred_element_type=jnp.float32,
    ) * (1.0 / math.sqrt(HD))
    row = i * bq + jax.lax.broadcasted_iota(jnp.int32, (bq, S), 0)
    col = jax.lax.broadcasted_iota(jnp.int32, (bq, S), 1)
    s = jnp.where(col > row, -1e30, s)
    m = jnp.max(s, axis=1, keepdims=True)
    e = jnp.exp(s - m)
    w = e / jnp.sum(e, axis=1, keepdims=True)
    o_ref[0] = jnp.dot(w, v_ref[0], preferred_element_type=jnp.float32)


def _attention(q, k, v, bq=256):
    return pl.pallas_call(
        functools.partial(_attn_kernel, bq=bq),
        grid=(H, S // bq),
        in_specs=[
            pl.BlockSpec((1, bq, HD), lambda h, i: (h, i, 0)),
            pl.BlockSpec((1, S, HD), lambda h, i: (h, 0, 0)),
            pl.BlockSpec((1, S, HD), lambda h, i: (h, 0, 0)),
        ],
        out_specs=pl.BlockSpec((1, bq, HD), lambda h, i: (h, i, 0)),
        out_shape=jax.ShapeDtypeStruct((H, S, HD), jnp.float32),
    )(q, k, v)


def _memattn_kernel(x_ref, g_ref, b_ref, w_ref, wb_ref, mk_ref, mv_ref, o_ref):
    xv = x_ref[...]
    mu = jnp.mean(xv, axis=1, keepdims=True)
    var = jnp.mean((xv - mu) ** 2, axis=1, keepdims=True)
    x2 = (xv - mu) / jnp.sqrt(var + 1e-5) * g_ref[...] + b_ref[...]
    q = jnp.dot(x2, w_ref[...], preferred_element_type=jnp.float32) + wb_ref[...]
    s = jax.lax.dot_general(
        q, mk_ref[...], (((1,), (1,)), ((), ())),
        preferred_element_type=jnp.float32,
    ) * (1.0 / math.sqrt(D))
    m = jnp.max(s, axis=1, keepdims=True)
    e = jnp.exp(s - m)
    a = e / jnp.sum(e, axis=1, keepdims=True)
    o_ref[...] = x2 + jnp.dot(a, mv_ref[...], preferred_element_type=jnp.float32)


def _memattn(x, g, b, ckw, ckb, mem_k, mem_v, bm=256):
    ml = mem_k.shape[0]
    return pl.pallas_call(
        _memattn_kernel,
        grid=(S // bm,),
        in_specs=[
            pl.BlockSpec((bm, D), lambda i: (i, 0)),
            pl.BlockSpec((1, D), lambda i: (0, 0)),
            pl.BlockSpec((1, D), lambda i: (0, 0)),
            pl.BlockSpec((D, D), lambda i: (0, 0)),
            pl.BlockSpec((1, D), lambda i: (0, 0)),
            pl.BlockSpec((ml, D), lambda i: (0, 0)),
            pl.BlockSpec((ml, D), lambda i: (0, 0)),
        ],
        out_specs=pl.BlockSpec((bm, D), lambda i: (i, 0)),
        out_shape=jax.ShapeDtypeStruct((S, D), jnp.float32),
    )(x, g.reshape(1, D), b.reshape(1, D), ckw, ckb.reshape(1, D), mem_k, mem_v)


def _expert_kernel(xf_ref, ti_ref, ts_ref, g_ref, b_ref, ew1_ref, eb1_ref,
                   ew2_ref, eb2_ref, o_ref, disp_ref, acc_ref):
    e = pl.program_id(0)
    kf = pl.program_id(1)

    @pl.when(jnp.logical_and(e == 0, kf == 0))
    def _():
        o_ref[...] = xf_ref[...]

    rows = jax.lax.broadcasted_iota(jnp.int32, (S, CAP), 0)
    oh = (rows == ti_ref[0]).astype(jnp.float32)

    @pl.when(kf == 0)
    def _():
        dv = jax.lax.dot_general(
            oh, xf_ref[...], (((0,), (0,)), ((), ())),
            precision=jax.lax.Precision.HIGHEST,
            preferred_element_type=jnp.float32,
        )
        mu = jnp.mean(dv, axis=1, keepdims=True)
        var = jnp.mean((dv - mu) ** 2, axis=1, keepdims=True)
        disp_ref[...] = (dv - mu) / jnp.sqrt(var + 1e-5) * g_ref[...] + b_ref[...]

    h1 = jnp.maximum(
        jnp.dot(disp_ref[...], ew1_ref[0], preferred_element_type=jnp.float32)
        + eb1_ref[0, 0], 0.0)
    contrib = jnp.dot(h1, ew2_ref[0], preferred_element_type=jnp.float32)

    @pl.when(kf == 0)
    def _():
        acc_ref[...] = contrib

    @pl.when(kf > 0)
    def _():
        acc_ref[...] = acc_ref[...] + contrib

    @pl.when(kf == KF - 1)
    def _():
        eo = acc_ref[...] + eb2_ref[0]
        o_ref[...] = o_ref[...] + jnp.dot(
            oh * ts_ref[0], eo, precision=jax.lax.Precision.HIGHEST,
            preferred_element_type=jnp.float32)


def _experts(xf, ti, ts, g3, b3, ew1, eb1, ew2, eb2):
    return pl.pallas_call(
        _expert_kernel,
        grid=(E, KF),
        in_specs=[
            pl.BlockSpec((S, D), lambda e, kf: (0, 0)),
            pl.BlockSpec((1, 1, CAP), lambda e, kf: (e, 0, 0)),
            pl.BlockSpec((1, 1, CAP), lambda e, kf: (e, 0, 0)),
            pl.BlockSpec((1, D), lambda e, kf: (0, 0)),
            pl.BlockSpec((1, D), lambda e, kf: (0, 0)),
            pl.BlockSpec((1, D, DFB), lambda e, kf: (e, 0, kf)),
            pl.BlockSpec((1, 1, 1, DFB), lambda e, kf: (e, kf, 0, 0)),
            pl.BlockSpec((1, DFB, D), lambda e, kf: (e, kf, 0)),
            pl.BlockSpec((1, 1, D), lambda e, kf: (e, 0, 0)),
        ],
        out_specs=pl.BlockSpec((S, D), lambda e, kf: (0, 0)),
        out_shape=jax.ShapeDtypeStruct((S, D), jnp.float32),
        scratch_shapes=[
            pltpu.VMEM((CAP, D), jnp.float32),
            pltpu.VMEM((CAP, D), jnp.float32),
        ],
    )(
        xf,
        ti.reshape(E, 1, CAP),
        ts.reshape(E, 1, CAP),
        g3.reshape(1, D),
        b3.reshape(1, D),
        ew1,
        eb1.reshape(E, KF, 1, DFB),
        ew2,
        eb2.reshape(E, 1, D),
    )


def _rec_kernel(ev_ref, od_ref, cq_ref, cpw_ref, cpb_ref, it_ref, fm_ref, o_ref):
    inv = 1.0 / math.sqrt(D)
    ev = ev_ref[...]
    od = od_ref[...]
    cq = cq_ref[...]
    se = jnp.sum(ev * cq, axis=1, keepdims=True) * inv
    so = jnp.sum(od * cq, axis=1, keepdims=True) * inv
    m = jnp.maximum(se, so)
    ae = jnp.exp(se - m)
    ao = jnp.exp(so - m)
    mix = (ae * ev + ao * od) / (ae + ao)
    comp = jnp.dot(mix, cpw_ref[...], preferred_element_type=jnp.float32) + cpb_ref[...]
    dec = jnp.dot(it_ref[...], comp, preferred_element_type=jnp.float32)
    d = dec - fm_ref[...]
    ssq = jnp.sum(jnp.sum(d * d, axis=1, keepdims=True), axis=0, keepdims=True)
    o_ref[...] = ssq * (1.0 / (MEM_LEN * D))


def _interp_mat():
    L, out_len = MEM_LEN // 2, MEM_LEN
    pos = (np.arange(out_len, dtype=np.float64) + 0.5) * L / out_len - 0.5
    pos = np.clip(pos, 0.0, L - 1.0)
    lo = np.floor(pos).astype(np.int32)
    hi = np.clip(lo + 1, 0, L - 1)
    w = (pos - lo).astype(np.float32)
    mat = np.zeros((out_len, L), np.float32)
    mat[np.arange(out_len), lo] += 1.0 - w
    mat[np.arange(out_len), hi] += w
    return jnp.asarray(mat)


def _rec_loss(fine_mem, cq, cpw, cpb):
    fm3 = fine_mem.reshape(MEM_LEN // 2, 2, D)
    return pl.pallas_call(
        _rec_kernel,
        out_shape=jax.ShapeDtypeStruct((1, 1), jnp.float32),
    )(fm3[:, 0, :], fm3[:, 1, :], cq.reshape(1, D), cpw, cpb.reshape(1, D),
      _interp_mat(), fine_mem)[0, 0]


def _imp_kernel(s_ref, o_ref):
    x = s_ref[...]
    colid = jax.lax.broadcasted_iota(jnp.int32, x.shape, 1)
    valid = colid < E
    xm = jnp.where(valid, x, -1e30)
    m = jnp.max(xm, axis=1, keepdims=True)
    ex = jnp.where(valid, jnp.exp(x - m), 0.0)
    p = ex / jnp.sum(ex, axis=1, keepdims=True)
    imp = jnp.sum(p, axis=0, keepdims=True)
    mean = jnp.sum(imp, axis=1, keepdims=True) / E
    dv = jnp.where(valid[:1, :], imp - mean, 0.0)
    var = jnp.sum(dv * dv, axis=1, keepdims=True) / (E - 1)
    o_ref[...] = var / (mean * mean + 1e-6)


def _imp_loss(scores_pad):
    return pl.pallas_call(
        _imp_kernel,
        out_shape=jax.ShapeDtypeStruct((1, 1), jnp.float32),
    )(scores_pad)[0, 0]


def kernel(x, fine_mem, cmem, params):
    p = params
    xf = x.reshape(S, D)

    wqkv = jnp.concatenate([p['Wq'], p['Wk'], p['Wv']], axis=1)
    qkv = _mm(xf, wqkv, ln=(p['g1'], p['b1']))
    q = qkv[:, :D].reshape(S, H, HD).transpose(1, 0, 2)
    k = qkv[:, D:2 * D].reshape(S, H, HD).transpose(1, 0, 2)
    v = qkv[:, 2 * D:].reshape(S, H, HD).transpose(1, 0, 2)
    ao = _attention(q, k, v).transpose(1, 0, 2).reshape(S, D)
    x1 = _mm(ao, p['Wo'], res=xf)

    fkv = _mm(fine_mem, jnp.concatenate([p['kpw'], p['vpw']], axis=1),
              jnp.concatenate([p['kpb'], p['vpb']]))
    ckv = _mm(cmem, jnp.concatenate([p['cmkw'], p['cmvw']], axis=1),
              jnp.concatenate([p['cmkb'], p['cmvb']]), bm=128)
    mem_k = jnp.concatenate([fkv[:, :D], ckv[:, :D]], axis=0)
    mem_v = jnp.concatenate([fkv[:, D:], ckv[:, D:]], axis=0)
    mem_out = _memattn(x1, p['g2'], p['b2'], p['ckw'], p['ckb'], mem_k, mem_v)
    x2 = _mm(mem_out, p['mpw'], p['mpb'], res=x1)

    rec = _rec_loss(fine_mem, p['cq'], p['cpw'], p['cpb'])

    hr = _mm(x2, p['rw1'], p['rb1'], ln=(p['g3'], p['b3']), act="relu")
    rw2p = jnp.pad(p['rw2'], ((0, 0), (0, 128 - E)))
    rb2p = jnp.pad(p['rb2'], (0, 128 - E))
    scores_pad = _mm(hr, rw2p, rb2p, bn=128)
    scores = scores_pad[:, :E]
    ts, ti = jax.lax.top_k(scores.T, CAP)
    out = _experts(x2, ti, ts, p['g3'], p['b3'],
                   p['ew1'], p['eb1'], p['ew2'], p['eb2'])

    imp = _imp_loss(scores_pad)
    aux = rec + imp
    return out.reshape(B, S, D), aux
